# baseline (speedup 1.0000x reference)
import functools

import jax
import jax.numpy as jnp
from jax.experimental import pallas as pl

N = 10000
GAMMA = 0.1
RB = 400
NRB = N // RB


def _spmm(ei, vals, x):
    msg = vals[:, None] * x[ei[1]]
    return jax.ops.segment_sum(msg, ei[0], num_segments=N)


def _outcomb_body(oa_ref, ok_ref, g_ref, gs_ref, s_ref, dk_ref, b_ref, o_ref):
    s = s_ref[...]
    dk = dk_ref[...]
    b = b_ref[...]
    gs = gs_ref[...]
    idt = g_ref[...] + gs + b[None, :]
    x = s * oa_ref[...] + (1.0 - s) * ok_ref[...] + gs + b[None, :] + GAMMA * dk * idt
    m = jnp.max(x, axis=1, keepdims=True)
    lse = jnp.log(jnp.sum(jnp.exp(x - m), axis=1, keepdims=True)) + m
    o_ref[...] = x - lse


def _outcomb(oa, ok, g, gs, s, dk, b, nclass):
    row = lambda i: (i, 0)
    return pl.pallas_call(
        _outcomb_body,
        grid=(NRB,),
        in_specs=[
            pl.BlockSpec((RB, nclass), row),
            pl.BlockSpec((RB, nclass), row),
            pl.BlockSpec((RB, nclass), row),
            pl.BlockSpec((RB, nclass), row),
            pl.BlockSpec((RB, 1), row),
            pl.BlockSpec((RB, 1), row),
            pl.BlockSpec((nclass,), lambda i: (0,)),
        ],
        out_specs=pl.BlockSpec((RB, nclass), row),
        out_shape=jax.ShapeDtypeStruct((N, nclass), jnp.float32),
    )(oa, ok, g, gs, s, dk, b)


def kernel(fea, adj_edge_index, adj_values, adj_knn_edge_index, adj_knn_values,
           W_in, Ws_in, b_in, W_mid, Ws_mid, b_mid, W_out, Ws_out, b_out,
           scores0, bias0, scores1, bias1, Dk0, Dbias0, Dk1, Dbias1):
    s_i = jax.nn.sigmoid(fea @ scores0 + bias0)
    Dk_i = fea @ Dk0 + Dbias0
    h = fea @ W_in
    hs = fea @ Ws_in
    a = _spmm(adj_edge_index, adj_values, h)
    k = _spmm(adj_knn_edge_index, adj_knn_values, h)
    x1 = s_i * a + (1.0 - s_i) * k + hs + b_in + GAMMA * Dk_i * (h + hs + b_in)
    x2 = _spmm(adj_edge_index, adj_values, x1 @ W_mid) + x1 @ Ws_mid + b_mid
    s_o = jax.nn.sigmoid(x2 @ scores1 + bias1)
    Dk_o = x2 @ Dk1 + Dbias1
    g = x2 @ W_out
    gs = x2 @ Ws_out
    oa = _spmm(adj_edge_index, adj_values, g)
    ok = _spmm(adj_knn_edge_index, adj_knn_values, g)
    return _outcomb(oa, ok, g, gs, s_o, Dk_o, b_out, g.shape[1])


# speedup vs baseline: 1.5864x; 1.5864x over previous
import functools

import jax
import jax.numpy as jnp
from jax import lax
from jax.experimental import pallas as pl
from jax.experimental.pallas import tpu as pltpu
from jax.experimental.pallas import tpu_sc as plsc

N = 10000
E = 160000
GAMMA = 0.1
RB = 400
NRB = N // RB

C = 128
NTILE = 16
FH = 128
E_PAD = 4096 * ((E + 4095) // 4096)
RPT = 624
TAIL = N - NTILE * RPT


def _make_spmm(edge_split):
    per_tile = E_PAD // (32 if edge_split else 16)
    n_chunks = per_tile // C
    t_rows = N if edge_split else 2 * N
    mesh = plsc.VectorSubcoreMesh(core_axis_name="c", subcore_axis_name="s",
                                  num_cores=2, num_subcores=NTILE)

    @functools.partial(
        pl.kernel,
        out_type=jax.ShapeDtypeStruct((2 * N, FH), jnp.float32),
        mesh=mesh,
        scratch_types=[
            pltpu.VMEM((C,), jnp.int32),
            pltpu.VMEM((1, C), jnp.int32),
            pltpu.VMEM((C, 16), jnp.float32),
            pltpu.VMEM((C, FH), jnp.float32),
            pltpu.VMEM_SHARED((N, FH), jnp.float32),
            pltpu.SemaphoreType.DMA,
        ],
    )
    def spmm(table_hbm, src_hbm, dst_hbm, vals_hbm, zeros_hbm, out_hbm,
             idx_src, idx_dst, vals_v, rows, acc, sem):
        c = lax.axis_index("c")
        s = lax.axis_index("s")
        base = s * per_tile
        if edge_split:
            base = base + c * (E_PAD // 2)

        pltpu.sync_copy(zeros_hbm, acc.at[pl.ds(s * RPT, RPT)])

        @pl.when(s == NTILE - 1)
        def _zero_tail():
            pltpu.sync_copy(zeros_hbm.at[pl.ds(0, TAIL)],
                            acc.at[pl.ds(NTILE * RPT, TAIL)])

        plsc.subcore_barrier()

        def chunk(j, carry):
            off = pl.multiple_of(base + j * C, C)
            pltpu.sync_copy(src_hbm.at[c, pl.ds(off, C)], idx_src)
            pltpu.sync_copy(dst_hbm.at[pl.ds(off, C)], idx_dst.at[0])
            pltpu.sync_copy(vals_hbm.at[pl.ds(off, C)], vals_v)
            pltpu.async_copy(table_hbm.at[idx_src], rows, sem).wait()
            for e in range(C):
                vs = vals_v[e, :]
                for f in range(FH // 16):
                    sl = pl.ds(f * 16, 16)
                    rows[e, sl] = rows[e, sl] * vs
            pltpu.sync_copy(rows, acc.at[idx_dst.at[0]], add=True)
            return carry

        lax.fori_loop(0, n_chunks, chunk, 0)
        plsc.subcore_barrier()
        pltpu.sync_copy(acc.at[pl.ds(s * RPT, RPT)],
                        out_hbm.at[pl.ds(c * N + s * RPT, RPT)])

        @pl.when(s == NTILE - 1)
        def _flush_tail():
            pltpu.sync_copy(acc.at[pl.ds(NTILE * RPT, TAIL)],
                            out_hbm.at[pl.ds(c * N + NTILE * RPT, TAIL)])

    return spmm


_SPMM_H = _make_spmm(False)
_SPMM_O = _make_spmm(True)


def _prep_edges(ei, vals):
    pad = E_PAD - E
    src = jnp.concatenate([ei[1], jnp.zeros((pad,), jnp.int32)])
    dst = jnp.concatenate([ei[0], jnp.zeros((pad,), jnp.int32)])
    v = jnp.concatenate([vals, jnp.zeros((pad,), jnp.float32)])
    vx = jnp.broadcast_to(v[:, None], (E_PAD, 16))
    src2c = jnp.stack([src, src + N])
    src2e = jnp.stack([src, src])
    return src2c, src2e, dst, vx


def _split(x):
    f = x.shape[1]
    return x.reshape(N, 2, f // 2).transpose(1, 0, 2).reshape(2 * N, f // 2)


def _unsplit(x2):
    fh = x2.shape[1]
    return x2.reshape(2, N, fh).transpose(1, 0, 2).reshape(N, 2 * fh)


def _spmm_h(table, edges, zeros):
    src2c, _, dst, v = edges
    return _unsplit(_SPMM_H(_split(table), src2c, dst, v, zeros))


def _spmm_o(table, edges, zeros):
    _, src2e, dst, v = edges
    nc = table.shape[1]
    t128 = jnp.pad(table, ((0, 0), (0, FH - nc)))
    out2 = _SPMM_O(t128, src2e, dst, v, zeros)
    return out2[:N, :nc] + out2[N:, :nc]


def _outcomb_body(oa_ref, ok_ref, g_ref, gs_ref, s_ref, dk_ref, b_ref, o_ref):
    s = s_ref[...]
    dk = dk_ref[...]
    b = b_ref[...]
    gs = gs_ref[...]
    idt = g_ref[...] + gs + b[None, :]
    x = s * oa_ref[...] + (1.0 - s) * ok_ref[...] + gs + b[None, :] + GAMMA * dk * idt
    m = jnp.max(x, axis=1, keepdims=True)
    lse = jnp.log(jnp.sum(jnp.exp(x - m), axis=1, keepdims=True)) + m
    o_ref[...] = x - lse


def _outcomb(oa, ok, g, gs, s, dk, b, nclass):
    row = lambda i: (i, 0)
    return pl.pallas_call(
        _outcomb_body,
        grid=(NRB,),
        in_specs=[
            pl.BlockSpec((RB, nclass), row),
            pl.BlockSpec((RB, nclass), row),
            pl.BlockSpec((RB, nclass), row),
            pl.BlockSpec((RB, nclass), row),
            pl.BlockSpec((RB, 1), row),
            pl.BlockSpec((RB, 1), row),
            pl.BlockSpec((nclass,), lambda i: (0,)),
        ],
        out_specs=pl.BlockSpec((RB, nclass), row),
        out_shape=jax.ShapeDtypeStruct((N, nclass), jnp.float32),
    )(oa, ok, g, gs, s, dk, b)


def kernel(fea, adj_edge_index, adj_values, adj_knn_edge_index, adj_knn_values,
           W_in, Ws_in, b_in, W_mid, Ws_mid, b_mid, W_out, Ws_out, b_out,
           scores0, bias0, scores1, bias1, Dk0, Dbias0, Dk1, Dbias1):
    ea = _prep_edges(adj_edge_index, adj_values)
    ek = _prep_edges(adj_knn_edge_index, adj_knn_values)
    zh = jnp.zeros((RPT, FH), jnp.float32)

    s_i = jax.nn.sigmoid(fea @ scores0 + bias0)
    Dk_i = fea @ Dk0 + Dbias0
    h = fea @ W_in
    hs = fea @ Ws_in
    a = _spmm_h(h, ea, zh)
    k = _spmm_h(h, ek, zh)
    x1 = s_i * a + (1.0 - s_i) * k + hs + b_in + GAMMA * Dk_i * (h + hs + b_in)
    x2 = _spmm_h(x1 @ W_mid, ea, zh) + x1 @ Ws_mid + b_mid
    s_o = jax.nn.sigmoid(x2 @ scores1 + bias1)
    Dk_o = x2 @ Dk1 + Dbias1
    g = x2 @ W_out
    gs = x2 @ Ws_out
    oa = _spmm_o(g, ea, zh)
    ok = _spmm_o(g, ek, zh)
    return _outcomb(oa, ok, g, gs, s_o, Dk_o, b_out, g.shape[1])


# speedup vs baseline: 2.2561x; 1.4221x over previous
import functools

import jax
import jax.numpy as jnp
from jax import lax
from jax.experimental import pallas as pl
from jax.experimental.pallas import tpu as pltpu
from jax.experimental.pallas import tpu_sc as plsc

N = 10000
E = 160000
GAMMA = 0.1
RB = 400
NRB = N // RB

C = 128
NTILE = 16
FH = 128
E_PAD = 4096 * ((E + 4095) // 4096)
RPT = 624
TAIL = N - NTILE * RPT


def _make_spmm(edge_split):
    per_tile = E_PAD // (32 if edge_split else 16)
    n_chunks = per_tile // C
    n_groups = n_chunks // 4
    mesh = plsc.VectorSubcoreMesh(core_axis_name="c", subcore_axis_name="s",
                                  num_cores=2, num_subcores=NTILE)

    @functools.partial(
        pl.kernel,
        out_type=jax.ShapeDtypeStruct((2 * N, FH), jnp.float32),
        mesh=mesh,
        scratch_types=[
            pltpu.VMEM((4, 2, C), jnp.int32),
            pltpu.VMEM((4 * C * 16,), jnp.float32),
            pltpu.VMEM((2, C, FH), jnp.float32),
            pltpu.VMEM_SHARED((N, FH), jnp.float32),
        ] + [pltpu.SemaphoreType.DMA] * 12,
    )
    def spmm(table_hbm, pk_hbm, vals_hbm, zeros_hbm, out_hbm,
             pk_v, vals_v, rows, acc,
             p0, p1, p2, p3, v0, v1, v2, v3, g0, g1, s0, s1):
        psem = [p0, p1, p2, p3]
        vsem = [v0, v1, v2, v3]
        gsem = [g0, g1]
        ssem = [s0, s1]
        c = lax.axis_index("c")
        s = lax.axis_index("s")
        base = s * per_tile
        if edge_split:
            base = base + c * (E_PAD // 2)

        VCH = C * 16

        def issue_pk(j, bq):
            off = pl.multiple_of(base + j * C, C)
            pltpu.async_copy(pk_hbm.at[c, :, pl.ds(off, C)], pk_v.at[bq], psem[bq])
            pltpu.async_copy(vals_hbm.at[pl.ds(off * 16, VCH)],
                             vals_v.at[pl.ds(bq * VCH, VCH)], vsem[bq])

        def drain_pk(bq):
            pltpu.make_async_copy(pk_hbm.at[c, :, pl.ds(0, C)],
                                  pk_v.at[bq], psem[bq]).wait()

        def drain_vals(bq):
            pltpu.make_async_copy(vals_hbm.at[pl.ds(0, VCH)],
                                  vals_v.at[pl.ds(bq * VCH, VCH)], vsem[bq]).wait()

        def issue_gather(bq, br):
            pltpu.async_copy(table_hbm.at[pk_v.at[bq, 0]], rows.at[br], gsem[br])

        def drain_gather(br):
            pltpu.make_async_copy(table_hbm.at[pl.ds(0, C)],
                                  rows.at[br], gsem[br]).wait()

        def issue_scatter(bq, br):
            pltpu.async_copy(rows.at[br], acc.at[pk_v.at[bq, 1]], ssem[br],
                             add=True)

        def drain_scatter(br):
            pltpu.make_async_copy(table_hbm.at[pl.ds(0, C)],
                                  rows.at[br], ssem[br]).wait()

        pltpu.sync_copy(zeros_hbm, acc.at[pl.ds(s * RPT, RPT)])

        @pl.when(s == NTILE - 1)
        def _zero_tail():
            pltpu.sync_copy(zeros_hbm.at[pl.ds(0, TAIL)],
                            acc.at[pl.ds(NTILE * RPT, TAIL)])

        plsc.subcore_barrier()

        issue_pk(0, 0)
        issue_pk(1, 1)
        drain_pk(0)
        issue_gather(0, 0)

        def group(gi, cy):
            for u in range(4):
                br = u % 2
                bq = u
                j = 4 * gi + u
                drain_gather(br)

                @pl.when(j + 1 < n_chunks)
                def _prep(j=j, br=br, bq=bq):
                    drain_pk((bq + 1) % 4)

                    @pl.when(j >= 1)
                    def _():
                        drain_scatter(1 - br)

                    issue_gather((bq + 1) % 4, 1 - br)

                drain_vals(bq)

                def scale(e, cy2, br=br, bq=bq):
                    vs = vals_v[pl.ds(bq * VCH + e * 16, 16)]
                    for f in range(FH // 16):
                        sl = pl.ds(f * 16, 16)
                        rows[br, e, sl] = rows[br, e, sl] * vs
                    return cy2

                lax.fori_loop(0, C, scale, 0)
                issue_scatter(bq, br)

                @pl.when(j + 2 < n_chunks)
                def _next(j=j, bq=bq):
                    issue_pk(j + 2, (bq + 2) % 4)

            return cy

        lax.fori_loop(0, n_groups, group, 0)
        drain_scatter(0)
        drain_scatter(1)
        plsc.subcore_barrier()
        pltpu.sync_copy(acc.at[pl.ds(s * RPT, RPT)],
                        out_hbm.at[pl.ds(c * N + s * RPT, RPT)])

        @pl.when(s == NTILE - 1)
        def _flush_tail():
            pltpu.sync_copy(acc.at[pl.ds(NTILE * RPT, TAIL)],
                            out_hbm.at[pl.ds(c * N + NTILE * RPT, TAIL)])

    return spmm


_SPMM_H = _make_spmm(False)
_SPMM_O = _make_spmm(True)


def _prep_edges(ei, vals):
    pad = E_PAD - E
    src = jnp.concatenate([ei[1], jnp.zeros((pad,), jnp.int32)])
    dst = jnp.concatenate([ei[0], jnp.zeros((pad,), jnp.int32)])
    v = jnp.concatenate([vals, jnp.zeros((pad,), jnp.float32)])
    vx = jnp.broadcast_to(v[:, None], (E_PAD, 16)).reshape(-1)
    pkc = jnp.stack([jnp.stack([src, dst]), jnp.stack([src + N, dst])])
    pke = jnp.stack([jnp.stack([src, dst])] * 2)
    return pkc, pke, vx


def _split(x):
    f = x.shape[1]
    return x.reshape(N, 2, f // 2).transpose(1, 0, 2).reshape(2 * N, f // 2)


def _unsplit(x2):
    fh = x2.shape[1]
    return x2.reshape(2, N, fh).transpose(1, 0, 2).reshape(N, 2 * fh)


def _spmm_h(table, edges, zeros):
    pkc, _, v = edges
    return _unsplit(_SPMM_H(_split(table), pkc, v, zeros))


def _spmm_o(table, edges, zeros):
    _, pke, v = edges
    nc = table.shape[1]
    t128 = jnp.pad(table, ((0, 0), (0, FH - nc)))
    out2 = _SPMM_O(t128, pke, v, zeros)
    return out2[:N, :nc] + out2[N:, :nc]


def _outcomb_body(oa_ref, ok_ref, g_ref, gs_ref, s_ref, dk_ref, b_ref, o_ref):
    s = s_ref[...]
    dk = dk_ref[...]
    b = b_ref[...]
    gs = gs_ref[...]
    idt = g_ref[...] + gs + b[None, :]
    x = s * oa_ref[...] + (1.0 - s) * ok_ref[...] + gs + b[None, :] + GAMMA * dk * idt
    m = jnp.max(x, axis=1, keepdims=True)
    lse = jnp.log(jnp.sum(jnp.exp(x - m), axis=1, keepdims=True)) + m
    o_ref[...] = x - lse


def _outcomb(oa, ok, g, gs, s, dk, b, nclass):
    row = lambda i: (i, 0)
    return pl.pallas_call(
        _outcomb_body,
        grid=(NRB,),
        in_specs=[
            pl.BlockSpec((RB, nclass), row),
            pl.BlockSpec((RB, nclass), row),
            pl.BlockSpec((RB, nclass), row),
            pl.BlockSpec((RB, nclass), row),
            pl.BlockSpec((RB, 1), row),
            pl.BlockSpec((RB, 1), row),
            pl.BlockSpec((nclass,), lambda i: (0,)),
        ],
        out_specs=pl.BlockSpec((RB, nclass), row),
        out_shape=jax.ShapeDtypeStruct((N, nclass), jnp.float32),
    )(oa, ok, g, gs, s, dk, b)


def kernel(fea, adj_edge_index, adj_values, adj_knn_edge_index, adj_knn_values,
           W_in, Ws_in, b_in, W_mid, Ws_mid, b_mid, W_out, Ws_out, b_out,
           scores0, bias0, scores1, bias1, Dk0, Dbias0, Dk1, Dbias1):
    ea = _prep_edges(adj_edge_index, adj_values)
    ek = _prep_edges(adj_knn_edge_index, adj_knn_values)
    zh = jnp.zeros((RPT, FH), jnp.float32)

    s_i = jax.nn.sigmoid(fea @ scores0 + bias0)
    Dk_i = fea @ Dk0 + Dbias0
    h = fea @ W_in
    hs = fea @ Ws_in
    a = _spmm_h(h, ea, zh)
    k = _spmm_h(h, ek, zh)
    x1 = s_i * a + (1.0 - s_i) * k + hs + b_in + GAMMA * Dk_i * (h + hs + b_in)
    x2 = _spmm_h(x1 @ W_mid, ea, zh) + x1 @ Ws_mid + b_mid
    s_o = jax.nn.sigmoid(x2 @ scores1 + bias1)
    Dk_o = x2 @ Dk1 + Dbias1
    g = x2 @ W_out
    gs = x2 @ Ws_out
    oa = _spmm_o(g, ea, zh)
    ok = _spmm_o(g, ek, zh)
    return _outcomb(oa, ok, g, gs, s_o, Dk_o, b_out, g.shape[1])
